# baseline (speedup 1.0000x reference)
import functools

import jax
import jax.numpy as jnp
from jax import lax
from jax.experimental import pallas as pl
from jax.experimental.pallas import tpu as pltpu
from jax.experimental.pallas import tpu_sc as plsc

B = 4096
M = 1024
D = 64
NC = 2
NS = 16
NW = NC * NS
L = 16
ROWS_W = B // NW
TILE = 32
NT = ROWS_W // TILE
MC = M // L
UNROLL = 4

_mesh = plsc.VectorSubcoreMesh(
    core_axis_name="c", subcore_axis_name="s", num_cores=NC, num_subcores=NS
)


def _s_mean_kernel(m_ref, s_ref):
    s_ref[...] = jnp.mean(m_ref[...], axis=1)


@functools.partial(
    pl.kernel,
    out_type=jax.ShapeDtypeStruct((B,), jnp.float32),
    mesh=_mesh,
    compiler_params=pltpu.CompilerParams(needs_layout_passes=False),
    scratch_types=[
        pltpu.VMEM((2, TILE, M), jnp.float32),
        pltpu.VMEM((M,), jnp.float32),
        pltpu.VMEM((L * L,), jnp.float32),
        pltpu.VMEM((ROWS_W,), jnp.float32),
        pltpu.SemaphoreType.DMA,
        pltpu.SemaphoreType.DMA,
    ],
)
def _sc_matvec(a_hbm, s_hbm, out_hbm, a_buf, s_buf, tr, out_buf, sem0, sem1):
    cid = lax.axis_index("c")
    sid = lax.axis_index("s")
    wid = sid * NC + cid
    base = wid * ROWS_W
    iota = lax.iota(jnp.int32, L)
    sems = (sem0, sem1)

    def a_copy(g, slot):
        return pltpu.make_async_copy(
            a_hbm.at[pl.ds(base + g * TILE, TILE), :],
            a_buf.at[slot],
            sems[slot],
        )

    a_copy(0, 0).start()
    if NT > 1:
        a_copy(1, 1).start()
    pltpu.sync_copy(s_hbm, s_buf)

    for g in range(NT):
        slot = g & 1
        a_copy(g, slot).wait()
        for half in range(TILE // L):

            def mac(i, accs, _slot=slot, _half=half):
                out = list(accs)
                for k in range(UNROLL):
                    off = (i * UNROLL + k) * L
                    vc = s_buf[pl.ds(off, L)]
                    for r in range(L):
                        out[r] = out[r] + a_buf[_slot, _half * L + r,
                                                pl.ds(off, L)] * vc
                return tuple(out)

            accs = lax.fori_loop(
                0, MC // UNROLL, mac,
                tuple(jnp.zeros((L,), jnp.float32) for _ in range(L)))
            for r in range(L):
                tr[pl.ds(r * L, L)] = accs[r]
            res = jnp.zeros((L,), jnp.float32)
            for c in range(L):
                res = res + plsc.load_gather(tr, [iota * L + c])
            out_buf[pl.ds((g * (TILE // L) + half) * L, L)] = res
        if g + 2 < NT:
            a_copy(g + 2, slot).start()

    pltpu.sync_copy(out_buf, out_hbm.at[pl.ds(base, ROWS_W)])


@jax.jit
def kernel(attention_weights, ability_means):
    s = pl.pallas_call(
        _s_mean_kernel,
        out_shape=jax.ShapeDtypeStruct((M,), jnp.float32),
    )(ability_means)
    return _sc_matvec(attention_weights, s)


# speedup vs baseline: 1.0668x; 1.0668x over previous
import functools

import jax
import jax.numpy as jnp
from jax import lax
from jax.experimental import pallas as pl
from jax.experimental.pallas import tpu as pltpu
from jax.experimental.pallas import tpu_sc as plsc

B = 4096
M = 1024
D = 64
NC = 2
NS = 16
NW = NC * NS
L = 16
ROWS_W = B // NW
TILE = 16
NT = ROWS_W // TILE
MC = M // L
UNROLL = 2
S_ROWS = M // NS

_mesh = plsc.VectorSubcoreMesh(
    core_axis_name="c", subcore_axis_name="s", num_cores=NC, num_subcores=NS
)


@functools.partial(
    pl.kernel,
    out_type=jax.ShapeDtypeStruct((B,), jnp.float32),
    mesh=_mesh,
    compiler_params=pltpu.CompilerParams(needs_layout_passes=False),
    scratch_types=[
        pltpu.VMEM((2, TILE, M), jnp.float32),
        pltpu.VMEM((M,), jnp.float32),
        pltpu.VMEM((S_ROWS, D), jnp.float32),
        pltpu.VMEM((L * L,), jnp.float32),
        pltpu.VMEM((ROWS_W,), jnp.float32),
        pltpu.VMEM_SHARED((M,), jnp.float32),
        pltpu.SemaphoreType.DMA,
        pltpu.SemaphoreType.DMA,
    ],
)
def _sc_matvec(a_hbm, m_hbm, out_hbm, a_buf, s_buf, m_buf, tr, out_buf,
               s_shared, sem0, sem1):
    cid = lax.axis_index("c")
    sid = lax.axis_index("s")
    wid = sid * NC + cid
    base = wid * ROWS_W
    iota = lax.iota(jnp.int32, L)
    sems = (sem0, sem1)

    def a_copy(g, slot):
        return pltpu.make_async_copy(
            a_hbm.at[pl.ds(base + g * TILE, TILE), :],
            a_buf.at[slot],
            sems[slot],
        )

    a_copy(0, 0).start()
    a_copy(1, 1).start()

    def gather_reduce(c, res):
        return res + plsc.load_gather(tr, [iota * L + c])

    pltpu.sync_copy(m_hbm.at[pl.ds(sid * S_ROWS, S_ROWS), :], m_buf)

    def s_chunk(j, _):
        def s_row(r, _):
            p = (m_buf[j * L + r, pl.ds(0, L)]
                 + m_buf[j * L + r, pl.ds(L, L)]
                 + m_buf[j * L + r, pl.ds(2 * L, L)]
                 + m_buf[j * L + r, pl.ds(3 * L, L)])
            tr[pl.ds(r * L, L)] = p
            return 0

        lax.fori_loop(0, L, s_row, 0)
        res = lax.fori_loop(0, L, gather_reduce, jnp.zeros((L,), jnp.float32))
        s_buf[pl.ds(sid * S_ROWS + j * L, L)] = res * (1.0 / D)
        return 0

    lax.fori_loop(0, S_ROWS // L, s_chunk, 0)
    pltpu.sync_copy(s_buf.at[pl.ds(sid * S_ROWS, S_ROWS)],
                    s_shared.at[pl.ds(sid * S_ROWS, S_ROWS)])
    plsc.subcore_barrier()
    pltpu.sync_copy(s_shared, s_buf)

    def do_group(g, slot):
        a_copy(g, slot).wait()

        def mac(i, accs):
            out = list(accs)
            for k in range(UNROLL):
                off = (i * UNROLL + k) * L
                vc = s_buf[pl.ds(off, L)]
                for r in range(L):
                    out[r] = out[r] + a_buf[slot, r, pl.ds(off, L)] * vc
            return tuple(out)

        accs = lax.fori_loop(
            0, MC // UNROLL, mac,
            tuple(jnp.zeros((L,), jnp.float32) for _ in range(L)))
        for r in range(L):
            tr[pl.ds(r * L, L)] = accs[r]
        res = lax.fori_loop(0, L, gather_reduce, jnp.zeros((L,), jnp.float32))
        out_buf[pl.ds(g * L, L)] = res

        @pl.when(g + 2 < NT)
        def _():
            a_copy(g + 2, slot).start()

    def pair(gg, _):
        do_group(2 * gg, 0)
        do_group(2 * gg + 1, 1)
        return 0

    lax.fori_loop(0, NT // 2, pair, 0)

    pltpu.sync_copy(out_buf, out_hbm.at[pl.ds(base, ROWS_W)])


@jax.jit
def kernel(attention_weights, ability_means):
    return _sc_matvec(attention_weights, ability_means)


# speedup vs baseline: 2.4935x; 2.3373x over previous
import functools

import jax
import jax.numpy as jnp
from jax.experimental import pallas as pl
from jax.experimental.pallas import tpu as pltpu

B = 4096
M = 1024
D = 64
BK = 512
NB = B // BK


def _matvec_body(a_ref, m_ref, o_ref):
    t = jnp.dot(a_ref[...], m_ref[...], preferred_element_type=jnp.float32)
    ones = jnp.ones((D, 1), jnp.float32)
    o_ref[...] = jnp.dot(t, ones, preferred_element_type=jnp.float32)[:, 0] * (1.0 / D)


@jax.jit
def kernel(attention_weights, ability_means):
    return pl.pallas_call(
        _matvec_body,
        grid=(NB,),
        in_specs=[
            pl.BlockSpec((BK, M), lambda i: (i, 0)),
            pl.BlockSpec((M, D), lambda i: (0, 0)),
        ],
        out_specs=pl.BlockSpec((BK,), lambda i: (i,)),
        out_shape=jax.ShapeDtypeStruct((B,), jnp.float32),
        compiler_params=pltpu.CompilerParams(
            dimension_semantics=("arbitrary",),
        ),
    )(attention_weights, ability_means)
